# baseline (speedup 1.0000x reference)
import functools

import jax
import jax.numpy as jnp
from jax import lax
from jax.experimental import pallas as pl
from jax.experimental.pallas import tpu as pltpu
from jax.experimental.pallas import tpu_sc as plsc

N = 10000
E = 320000
C = 128

NPAD = 10240
TRASH = NPAD - 1
NCORES = 2
NSUB = 16
NW = NCORES * NSUB
EPW = E // NW
CHUNK = 32
NCHUNK = 314
EPWP = NCHUNK * CHUNK
NITER = NCHUNK // 2
PCH = 3 * CHUNK
RPW = NPAD // NSUB
L = 16


def _prep_body(x_ref, w1_ref, de_ref, w2_ref, ts_ref, td_ref, dv_ref):
    xb = x_ref[...]
    w1 = w1_ref[...]
    ts_ref[:, :C] = jnp.dot(xb, w1[:C], preferred_element_type=jnp.float32)
    ts_ref[:, C:] = xb
    td_ref[...] = jnp.dot(xb, w1[C:], preferred_element_type=jnp.float32)
    dv = jnp.dot(de_ref[...], w2_ref[...][C:], preferred_element_type=jnp.float32)
    dv_ref[...] = jnp.concatenate([dv, jnp.zeros((12, 1), jnp.float32)], axis=0)


def _precompute(x, W1, W2, dist_emb):
    blk = 1000
    grid = (N // blk,)
    return pl.pallas_call(
        _prep_body,
        grid=grid,
        in_specs=[
            pl.BlockSpec((blk, C), lambda i: (i, 0)),
            pl.BlockSpec((2 * C, C), lambda i: (0, 0)),
            pl.BlockSpec((20, C), lambda i: (0, 0)),
            pl.BlockSpec((2 * C, 1), lambda i: (0, 0)),
        ],
        out_specs=[
            pl.BlockSpec((blk, 2 * C), lambda i: (i, 0)),
            pl.BlockSpec((blk, C), lambda i: (i, 0)),
            pl.BlockSpec((32, 1), lambda i: (0, 0)),
        ],
        out_shape=[
            jax.ShapeDtypeStruct((N, 2 * C), jnp.float32),
            jax.ShapeDtypeStruct((N, C), jnp.float32),
            jax.ShapeDtypeStruct((32, 1), jnp.float32),
        ],
    )(x, W1, dist_emb, W2)


def _sc_body(ts_hbm, td_hbm, ep_hbm, w2a_hbm, dval_hbm,
             agg_out, cnt_out,
             rows_sA, rows_sB, rows_dA, rows_dB, wbufA, wbufB,
             attbA, attbB, ibufA, ibufB, dscatA, dscatB,
             w2a_v, dval_v, agg_sh, cnt_sh,
             gsemA, gsemB, ssemA, ssemB, isemA, isemB):
    cid = lax.axis_index("c")
    sid = lax.axis_index("s")
    wid = cid * NSUB + sid
    gbase = wid * NCHUNK

    zeros16 = jnp.zeros((L,), jnp.float32)

    def zrow(r, carry):
        for j in range(C // L):
            wbufA[r, j * L:(j + 1) * L] = zeros16
        attbA[r, 0:L] = zeros16
        return carry
    lax.fori_loop(0, CHUNK, zrow, 0)
    for k in range(RPW // CHUNK):
        off = sid * RPW + k * CHUNK
        pltpu.sync_copy(wbufA, agg_sh.at[pl.ds(off, CHUNK)])
        pltpu.sync_copy(attbA, cnt_sh.at[pl.ds(off, CHUNK)])

    pltpu.sync_copy(w2a_hbm, w2a_v)
    pltpu.sync_copy(dval_hbm, dval_v)
    plsc.subcore_barrier()

    w2a_vecs = [plsc.bitcast(w2a_v[j * L:(j + 1) * L], jnp.bfloat16)
                for j in range(C // (2 * L))]
    dv_lo = dval_v[0:L]
    dv_hi = dval_v[L:2 * L]
    dval_sc = [dv_lo[b] for b in range(L)] + [dv_hi[b] for b in range(4)]
    iota16 = lax.iota(jnp.int32, L)

    def idx_issue(c, ibuf, isem):
        off = pl.multiple_of((gbase + c) * PCH, PCH)
        pltpu.async_copy(ep_hbm.at[pl.ds(off, PCH)], ibuf, isem)

    def idx_wait(ibuf, isem):
        pltpu.make_async_copy(ep_hbm.at[pl.ds(0, PCH)], ibuf, isem).wait()

    def gather_issue(ibuf, rs, rd, gsem):
        pltpu.async_copy(ts_hbm.at[ibuf.at[pl.ds(0, CHUNK)]], rs, gsem)
        pltpu.async_copy(td_hbm.at[ibuf.at[pl.ds(CHUNK, CHUNK)]], rd, gsem)

    def gather_wait(rs, rd, gsem):
        pltpu.make_async_copy(ts_hbm.at[pl.ds(0, CHUNK)], rs, gsem).wait()
        pltpu.make_async_copy(td_hbm.at[pl.ds(0, CHUNK)], rd, gsem).wait()

    def scatter_issue(wb, ab, dscat, ssem):
        pltpu.async_copy(wb, agg_sh.at[dscat], ssem, add=True)
        pltpu.async_copy(ab, cnt_sh.at[dscat], ssem, add=True)

    def scatter_wait(wb, ab, ssem):
        pltpu.make_async_copy(
            ts_hbm.at[pl.ds(0, CHUNK), pl.ds(0, C)], wb, ssem).wait()
        pltpu.make_async_copy(
            ts_hbm.at[pl.ds(0, CHUNK), pl.ds(0, L)], ab, ssem).wait()

    def compute_chunk(ibuf, rs, rd, wb, ab, dscat):
        for j in range(CHUNK // L):
            dscat[j * L:(j + 1) * L] = ibuf[CHUNK + j * L:CHUNK + (j + 1) * L]

        def group(g, carry):
            s_sc = []
            for ee in range(L):
                e = g * L + ee
                acc = zeros16
                for j in range(C // (2 * L)):
                    ga = plsc.bitcast(rs[e, j * L:(j + 1) * L], jnp.bfloat16)
                    gb = plsc.bitcast(rd[e, j * L:(j + 1) * L], jnp.bfloat16)
                    h = ga + gb
                    lr = jnp.maximum(h, h * jnp.bfloat16(0.2))
                    p = lr * w2a_vecs[j]
                    pa, pb = plsc.unpack(p, format=plsc.PackFormat.INTERLEAVED)
                    acc = acc + pa + pb
                s_sc.append(jnp.sum(acc))
            logits = jnp.full((L,), s_sc[0], jnp.float32)
            for ee in range(1, L):
                logits = jnp.where(iota16 == ee, s_sc[ee], logits)
            db = ibuf[pl.ds(2 * CHUNK + g * L, L)] // 50
            dv = jnp.full((L,), dval_sc[19], jnp.float32)
            for b in range(19):
                dv = jnp.where(db == b, dval_sc[b], dv)
            logits = logits + dv
            sg = 1.0 / (1.0 + jnp.exp(-logits))
            att = jnp.exp(sg)
            for ee in range(L):
                e = g * L + ee
                attbc = jnp.full((L,), att[ee], jnp.float32)
                xoff = C // 2
                for j in range(C // L):
                    wb[e, j * L:(j + 1) * L] = (
                        rs[e, xoff + j * L:xoff + (j + 1) * L] * attbc)
                ab[e, 0:L] = attbc
            return carry
        lax.fori_loop(0, CHUNK // L, group, 0)

    pltpu.sync_copy(ep_hbm.at[pl.ds(pl.multiple_of(gbase * PCH, PCH), PCH)],
                    ibufA)
    gather_issue(ibufA, rows_sA, rows_dA, gsemA)
    idx_issue(1, ibufB, isemB)

    def pipe(k, carry):
        idx_wait(ibufB, isemB)
        gather_issue(ibufB, rows_sB, rows_dB, gsemB)
        gather_wait(rows_sA, rows_dA, gsemA)

        @pl.when(k > 0)
        def _():
            scatter_wait(wbufA, attbA, ssemA)
        compute_chunk(ibufA, rows_sA, rows_dA, wbufA, attbA, dscatA)
        scatter_issue(wbufA, attbA, dscatA, ssemA)

        @pl.when(k < NITER - 1)
        def _():
            idx_issue(2 * k + 2, ibufA, isemA)

        @pl.when(k < NITER - 1)
        def _():
            idx_wait(ibufA, isemA)
            gather_issue(ibufA, rows_sA, rows_dA, gsemA)
        gather_wait(rows_sB, rows_dB, gsemB)

        @pl.when(k > 0)
        def _():
            scatter_wait(wbufB, attbB, ssemB)
        compute_chunk(ibufB, rows_sB, rows_dB, wbufB, attbB, dscatB)
        scatter_issue(wbufB, attbB, dscatB, ssemB)

        @pl.when(k < NITER - 1)
        def _():
            idx_issue(2 * k + 3, ibufB, isemB)
        return carry
    lax.fori_loop(0, NITER, pipe, 0)

    scatter_wait(wbufA, attbA, ssemA)
    scatter_wait(wbufB, attbB, ssemB)

    plsc.subcore_barrier()
    out_off = sid * RPW
    pltpu.sync_copy(agg_sh.at[pl.ds(out_off, RPW)],
                    agg_out.at[cid, pl.ds(out_off, RPW)])
    pltpu.sync_copy(cnt_sh.at[pl.ds(out_off, RPW)],
                    cnt_out.at[cid, pl.ds(out_off, RPW)])


def _sc_edges(table_src, table_dst, epack, w2a, dval):
    mesh = plsc.VectorSubcoreMesh(core_axis_name="c", subcore_axis_name="s",
                                  num_cores=NCORES)
    f = pl.kernel(
        _sc_body,
        out_type=[
            jax.ShapeDtypeStruct((NCORES, NPAD, C), jnp.float32),
            jax.ShapeDtypeStruct((NCORES, NPAD, L), jnp.float32),
        ],
        mesh=mesh,
        compiler_params=pltpu.CompilerParams(needs_layout_passes=False,
                                             use_tc_tiling_on_sc=False),
        scratch_types=[
            pltpu.VMEM((CHUNK, 3 * C // 2), jnp.float32),
            pltpu.VMEM((CHUNK, 3 * C // 2), jnp.float32),
            pltpu.VMEM((CHUNK, C // 2), jnp.float32),
            pltpu.VMEM((CHUNK, C // 2), jnp.float32),
            pltpu.VMEM((CHUNK, C), jnp.float32),
            pltpu.VMEM((CHUNK, C), jnp.float32),
            pltpu.VMEM((CHUNK, L), jnp.float32),
            pltpu.VMEM((CHUNK, L), jnp.float32),
            pltpu.VMEM((PCH,), jnp.int32),
            pltpu.VMEM((PCH,), jnp.int32),
            pltpu.VMEM((CHUNK,), jnp.int32),
            pltpu.VMEM((CHUNK,), jnp.int32),
            pltpu.VMEM((C // 2,), jnp.float32),
            pltpu.VMEM((32,), jnp.float32),
            pltpu.VMEM_SHARED((NPAD, C), jnp.float32),
            pltpu.VMEM_SHARED((NPAD, L), jnp.float32),
            pltpu.SemaphoreType.DMA,
            pltpu.SemaphoreType.DMA,
            pltpu.SemaphoreType.DMA,
            pltpu.SemaphoreType.DMA,
            pltpu.SemaphoreType.DMA,
            pltpu.SemaphoreType.DMA,
        ],
    )
    return f(table_src, table_dst, epack, w2a, dval)


def _pack_edges(edge_index, distances):
    src = edge_index[0].reshape(NW, EPW)
    dst = edge_index[1].reshape(NW, EPW)
    dist = distances.reshape(NW, EPW)
    padn = EPWP - EPW
    src = jnp.pad(src, ((0, 0), (0, padn)))
    dst = jnp.pad(dst, ((0, 0), (0, padn)), constant_values=TRASH)
    dist = jnp.pad(dist, ((0, 0), (0, padn)))
    rec = jnp.concatenate([src.reshape(NW, NCHUNK, CHUNK),
                           dst.reshape(NW, NCHUNK, CHUNK),
                           dist.reshape(NW, NCHUNK, CHUNK)], axis=2)
    return rec.reshape(-1)


def _fin_body(agg_ref, cnt_ref, out_ref):
    a = agg_ref[0]
    c = cnt_ref[0, :, 0:1]
    for k in range(1, NCORES):
        a = a + agg_ref[k]
        c = c + cnt_ref[k, :, 0:1]
    out_ref[...] = jnp.maximum(a / (c + 1e-6), 0.0)


def _finalize(agg, cnt):
    blk = 1280
    grid = (NPAD // blk,)
    return pl.pallas_call(
        _fin_body,
        grid=grid,
        in_specs=[
            pl.BlockSpec((NCORES, blk, C), lambda i: (0, i, 0)),
            pl.BlockSpec((NCORES, blk, L), lambda i: (0, i, 0)),
        ],
        out_specs=pl.BlockSpec((blk, C), lambda i: (i, 0)),
        out_shape=jax.ShapeDtypeStruct((NPAD, C), jnp.float32),
    )(agg, cnt)


def _bf16_pack(a):
    ab = a.astype(jnp.bfloat16)
    return lax.bitcast_convert_type(
        ab.reshape(*ab.shape[:-1], ab.shape[-1] // 2, 2), jnp.float32)


def kernel(x, edge_index, distances, W1, W2, dist_emb):
    table_src, table_dst, dval = _precompute(x, W1, W2, dist_emb)
    epack = _pack_edges(edge_index, distances)
    table_m = jnp.concatenate([_bf16_pack(table_src[:, :C]),
                               table_src[:, C:]], axis=1)
    xb_pk = _bf16_pack(table_dst)
    w2a_pk = _bf16_pack(W2[:C, 0])
    agg, cnt = _sc_edges(table_m, xb_pk, epack, w2a_pk, dval[:, 0])
    out = _finalize(agg, cnt)
    return out[:N]


# speedup vs baseline: 1.6470x; 1.6470x over previous
import functools

import jax
import jax.numpy as jnp
from jax import lax
from jax.experimental import pallas as pl
from jax.experimental.pallas import tpu as pltpu
from jax.experimental.pallas import tpu_sc as plsc

N = 10000
E = 320000
C = 128

NPAD = 10240
TRASH = NPAD - 1
NCORES = 2
NSUB = 16
NW = NCORES * NSUB
EPW = E // NW
CHUNK = 32
NCHUNK = 314
EPWP = NCHUNK * CHUNK
NITER = NCHUNK // 2
PCH = 3 * CHUNK
RPW = NPAD // NSUB
L = 16


def _prep_body(x_ref, w1_ref, ts_ref, td_ref):
    xb = x_ref[...]
    w1 = w1_ref[...]
    ts_ref[:, :C] = jnp.dot(xb, w1[:C], preferred_element_type=jnp.float32)
    ts_ref[:, C:] = xb
    td_ref[...] = jnp.dot(xb, w1[C:], preferred_element_type=jnp.float32)


def _precompute(x, W1):
    blk = 1000
    grid = (N // blk,)
    return pl.pallas_call(
        _prep_body,
        grid=grid,
        in_specs=[
            pl.BlockSpec((blk, C), lambda i: (i, 0)),
            pl.BlockSpec((2 * C, C), lambda i: (0, 0)),
        ],
        out_specs=[
            pl.BlockSpec((blk, 2 * C), lambda i: (i, 0)),
            pl.BlockSpec((blk, C), lambda i: (i, 0)),
        ],
        out_shape=[
            jax.ShapeDtypeStruct((N, 2 * C), jnp.float32),
            jax.ShapeDtypeStruct((N, C), jnp.float32),
        ],
    )(x, W1)


DBLK = 128


def _dve_body(d_ref, de_ref, w2_ref, out_ref):
    dv20 = jnp.dot(de_ref[...], w2_ref[...][C:],
                   preferred_element_type=jnp.float32)
    db = d_ref[...] // 50
    val = jnp.full(db.shape, dv20[19, 0], jnp.float32)
    for b in range(19):
        val = jnp.where(db == b, dv20[b, 0], val)
    out_ref[...] = val


def _dval_edges(dist2, dist_emb, W2):
    rows = E // DBLK
    return pl.pallas_call(
        _dve_body,
        grid=(1,),
        in_specs=[
            pl.BlockSpec((rows, DBLK), lambda i: (0, 0)),
            pl.BlockSpec((20, C), lambda i: (0, 0)),
            pl.BlockSpec((2 * C, 1), lambda i: (0, 0)),
        ],
        out_specs=pl.BlockSpec((rows, DBLK), lambda i: (0, 0)),
        out_shape=jax.ShapeDtypeStruct((rows, DBLK), jnp.float32),
    )(dist2, dist_emb, W2)


def _sc_body(ts_hbm, td_hbm, ep_hbm, w2a_hbm,
             agg_out, cnt_out,
             rows_sA, rows_sB, rows_dA, rows_dB, wbufA, wbufB,
             attbA, attbB, ibufA, ibufB, dscatA, dscatB,
             w2a_v, agg_sh, cnt_sh,
             gsemA, gsemB, ssemA, ssemB, isemA, isemB):
    cid = lax.axis_index("c")
    sid = lax.axis_index("s")
    wid = cid * NSUB + sid
    gbase = wid * NCHUNK

    zeros16 = jnp.zeros((L,), jnp.float32)

    def zrow(r, carry):
        for j in range(C // L):
            wbufA[r, j * L:(j + 1) * L] = zeros16
        attbA[r, 0:L] = zeros16
        return carry
    lax.fori_loop(0, CHUNK, zrow, 0)
    for k in range(RPW // CHUNK):
        off = sid * RPW + k * CHUNK
        pltpu.sync_copy(wbufA, agg_sh.at[pl.ds(off, CHUNK)])
        pltpu.sync_copy(attbA, cnt_sh.at[pl.ds(off, CHUNK)])

    pltpu.sync_copy(w2a_hbm, w2a_v)
    plsc.subcore_barrier()

    w2a_vecs = [plsc.bitcast(w2a_v[j * L:(j + 1) * L], jnp.bfloat16)
                for j in range(C // (2 * L))]
    iota16 = lax.iota(jnp.int32, L)

    def idx_issue(c, ibuf, isem):
        off = pl.multiple_of((gbase + c) * PCH, PCH)
        pltpu.async_copy(ep_hbm.at[pl.ds(off, PCH)], ibuf, isem)

    def idx_wait(ibuf, isem):
        pltpu.make_async_copy(ep_hbm.at[pl.ds(0, PCH)], ibuf, isem).wait()

    def gather_issue(ibuf, rs, rd, gsem):
        pltpu.async_copy(ts_hbm.at[ibuf.at[pl.ds(0, CHUNK)]], rs, gsem)
        pltpu.async_copy(td_hbm.at[ibuf.at[pl.ds(CHUNK, CHUNK)]], rd, gsem)

    def gather_wait(rs, rd, gsem):
        pltpu.make_async_copy(ts_hbm.at[pl.ds(0, CHUNK)], rs, gsem).wait()
        pltpu.make_async_copy(td_hbm.at[pl.ds(0, CHUNK)], rd, gsem).wait()

    def scatter_issue(wb, ab, dscat, ssem):
        pltpu.async_copy(wb, agg_sh.at[dscat], ssem, add=True)
        pltpu.async_copy(ab, cnt_sh.at[dscat], ssem, add=True)

    def scatter_wait(wb, ab, ssem):
        pltpu.make_async_copy(
            ts_hbm.at[pl.ds(0, CHUNK), pl.ds(0, C)], wb, ssem).wait()
        pltpu.make_async_copy(
            ts_hbm.at[pl.ds(0, CHUNK), pl.ds(0, L)], ab, ssem).wait()

    def compute_chunk(ibuf, rs, rd, wb, ab, dscat):
        for j in range(CHUNK // L):
            dscat[j * L:(j + 1) * L] = ibuf[CHUNK + j * L:CHUNK + (j + 1) * L]

        xoff = C // 2
        for g in range(CHUNK // L):
            s_sc = []
            for ee in range(L):
                e = g * L + ee
                acc = zeros16
                for j in range(C // (2 * L)):
                    ga = plsc.bitcast(rs[e, j * L:(j + 1) * L], jnp.bfloat16)
                    gb = plsc.bitcast(rd[e, j * L:(j + 1) * L], jnp.bfloat16)
                    h = ga + gb
                    lr = jnp.maximum(h, h * jnp.bfloat16(0.2))
                    p = lr * w2a_vecs[j]
                    pa, pb = plsc.unpack(p, format=plsc.PackFormat.INTERLEAVED)
                    acc = acc + pa + pb
                s_sc.append(jnp.sum(acc))
            logits = jnp.full((L,), s_sc[0], jnp.float32)
            for ee in range(1, L):
                logits = jnp.where(iota16 == ee, s_sc[ee], logits)
            dv = plsc.bitcast(
                ibuf[2 * CHUNK + g * L:2 * CHUNK + (g + 1) * L], jnp.float32)
            logits = logits + dv
            sg = 1.0 / (1.0 + jnp.exp(-logits))
            att = jnp.exp(sg)
            for ee in range(L):
                e = g * L + ee
                attbc = jnp.full((L,), att[ee], jnp.float32)
                for j in range(C // L):
                    wb[e, j * L:(j + 1) * L] = (
                        rs[e, xoff + j * L:xoff + (j + 1) * L] * attbc)
                ab[e, 0:L] = attbc

    pltpu.sync_copy(ep_hbm.at[pl.ds(pl.multiple_of(gbase * PCH, PCH), PCH)],
                    ibufA)
    gather_issue(ibufA, rows_sA, rows_dA, gsemA)
    idx_issue(1, ibufB, isemB)

    def pipe(k, carry):
        idx_wait(ibufB, isemB)
        gather_issue(ibufB, rows_sB, rows_dB, gsemB)
        gather_wait(rows_sA, rows_dA, gsemA)

        @pl.when(k > 0)
        def _():
            scatter_wait(wbufA, attbA, ssemA)
        compute_chunk(ibufA, rows_sA, rows_dA, wbufA, attbA, dscatA)
        scatter_issue(wbufA, attbA, dscatA, ssemA)

        @pl.when(k < NITER - 1)
        def _():
            idx_issue(2 * k + 2, ibufA, isemA)

        @pl.when(k < NITER - 1)
        def _():
            idx_wait(ibufA, isemA)
            gather_issue(ibufA, rows_sA, rows_dA, gsemA)
        gather_wait(rows_sB, rows_dB, gsemB)

        @pl.when(k > 0)
        def _():
            scatter_wait(wbufB, attbB, ssemB)
        compute_chunk(ibufB, rows_sB, rows_dB, wbufB, attbB, dscatB)
        scatter_issue(wbufB, attbB, dscatB, ssemB)

        @pl.when(k < NITER - 1)
        def _():
            idx_issue(2 * k + 3, ibufB, isemB)
        return carry
    lax.fori_loop(0, NITER, pipe, 0)

    scatter_wait(wbufA, attbA, ssemA)
    scatter_wait(wbufB, attbB, ssemB)

    plsc.subcore_barrier()
    out_off = sid * RPW
    pltpu.sync_copy(agg_sh.at[pl.ds(out_off, RPW)],
                    agg_out.at[cid, pl.ds(out_off, RPW)])
    pltpu.sync_copy(cnt_sh.at[pl.ds(out_off, RPW)],
                    cnt_out.at[cid, pl.ds(out_off, RPW)])


def _sc_edges(table_src, table_dst, epack, w2a):
    mesh = plsc.VectorSubcoreMesh(core_axis_name="c", subcore_axis_name="s",
                                  num_cores=NCORES)
    f = pl.kernel(
        _sc_body,
        out_type=[
            jax.ShapeDtypeStruct((NCORES, NPAD, C), jnp.float32),
            jax.ShapeDtypeStruct((NCORES, NPAD, L), jnp.float32),
        ],
        mesh=mesh,
        compiler_params=pltpu.CompilerParams(needs_layout_passes=False,
                                             use_tc_tiling_on_sc=False),
        scratch_types=[
            pltpu.VMEM((CHUNK, 3 * C // 2), jnp.float32),
            pltpu.VMEM((CHUNK, 3 * C // 2), jnp.float32),
            pltpu.VMEM((CHUNK, C // 2), jnp.float32),
            pltpu.VMEM((CHUNK, C // 2), jnp.float32),
            pltpu.VMEM((CHUNK, C), jnp.float32),
            pltpu.VMEM((CHUNK, C), jnp.float32),
            pltpu.VMEM((CHUNK, L), jnp.float32),
            pltpu.VMEM((CHUNK, L), jnp.float32),
            pltpu.VMEM((PCH,), jnp.int32),
            pltpu.VMEM((PCH,), jnp.int32),
            pltpu.VMEM((CHUNK,), jnp.int32),
            pltpu.VMEM((CHUNK,), jnp.int32),
            pltpu.VMEM((C // 2,), jnp.float32),
            pltpu.VMEM_SHARED((NPAD, C), jnp.float32),
            pltpu.VMEM_SHARED((NPAD, L), jnp.float32),
            pltpu.SemaphoreType.DMA,
            pltpu.SemaphoreType.DMA,
            pltpu.SemaphoreType.DMA,
            pltpu.SemaphoreType.DMA,
            pltpu.SemaphoreType.DMA,
            pltpu.SemaphoreType.DMA,
        ],
    )
    return f(table_src, table_dst, epack, w2a)


def _pack_edges(edge_index, dve_i32):
    src = edge_index[0].reshape(NW, EPW)
    dst = edge_index[1].reshape(NW, EPW)
    dve = dve_i32.reshape(NW, EPW)
    padn = EPWP - EPW
    src = jnp.pad(src, ((0, 0), (0, padn)))
    dst = jnp.pad(dst, ((0, 0), (0, padn)), constant_values=TRASH)
    dve = jnp.pad(dve, ((0, 0), (0, padn)))
    rec = jnp.concatenate([src.reshape(NW, NCHUNK, CHUNK),
                           dst.reshape(NW, NCHUNK, CHUNK),
                           dve.reshape(NW, NCHUNK, CHUNK)], axis=2)
    return rec.reshape(-1)


def _fin_body(agg_ref, cnt_ref, out_ref):
    a = agg_ref[0]
    c = cnt_ref[0, :, 0:1]
    for k in range(1, NCORES):
        a = a + agg_ref[k]
        c = c + cnt_ref[k, :, 0:1]
    out_ref[...] = jnp.maximum(a / (c + 1e-6), 0.0)


def _finalize(agg, cnt):
    blk = 1280
    grid = (NPAD // blk,)
    return pl.pallas_call(
        _fin_body,
        grid=grid,
        in_specs=[
            pl.BlockSpec((NCORES, blk, C), lambda i: (0, i, 0)),
            pl.BlockSpec((NCORES, blk, L), lambda i: (0, i, 0)),
        ],
        out_specs=pl.BlockSpec((blk, C), lambda i: (i, 0)),
        out_shape=jax.ShapeDtypeStruct((NPAD, C), jnp.float32),
    )(agg, cnt)


def _bf16_pack(a):
    ab = a.astype(jnp.bfloat16)
    return lax.bitcast_convert_type(
        ab.reshape(*ab.shape[:-1], ab.shape[-1] // 2, 2), jnp.float32)


def kernel(x, edge_index, distances, W1, W2, dist_emb):
    table_src, table_dst = _precompute(x, W1)
    dve = _dval_edges(distances.reshape(E // DBLK, DBLK), dist_emb, W2)
    dve_i32 = lax.bitcast_convert_type(dve.reshape(E), jnp.int32)
    epack = _pack_edges(edge_index, dve_i32)
    table_m = jnp.concatenate([_bf16_pack(table_src[:, :C]),
                               table_src[:, C:]], axis=1)
    xb_pk = _bf16_pack(table_dst)
    w2a_pk = _bf16_pack(W2[:C, 0])
    agg, cnt = _sc_edges(table_m, xb_pk, epack, w2a_pk)
    out = _finalize(agg, cnt)
    return out[:N]
